# Initial kernel scaffold; baseline (speedup 1.0000x reference)
import jax
import jax.numpy as jnp
from jax.experimental import pallas as pl


def kernel(w, c, p, word_vectors, char_table, W_proj, Wt0, bt0, Wg0, bg0, Wt1, bt1, Wg1, bg1):
    raise NotImplementedError("write your pallas kernel here")


# speedup vs baseline: 3.0099x; 3.0099x over previous
import functools

import jax
import jax.numpy as jnp
from jax import lax
from jax.experimental import pallas as pl
from jax.experimental.pallas import tpu as pltpu
from jax.experimental.pallas import tpu_sc as plsc

B, LW, LC = 64, 512, 2048
DW, DC, H = 300, 64, 128
DCP = 128

NC, NS = 2, 16
NW = NC * NS

ROWS_PER_W = B * LW // NW
WCHUNK = 128
NWCHUNK = ROWS_PER_W // WCHUNK
BPW = B // NW
CCHUNK = 128
NCCHUNK = LC // CCHUNK
ZROWS = LW // NS

_f32 = jnp.float32
_i32 = jnp.int32


DW_SPLIT = ((0, 128), (128, 128), (256, DW - 256))


def _sc_body(w_hbm, c_hbm, p_hbm, table_hbm, ttail_hbm, ctab_hbm,
             embw0_hbm, embw1_hbm, embw2_hbm, acc_hbm,
             widx, wbuf0, wbuf1, wbuf2, cbuf, pbuf, gbuf, zbuf, acc_sh, sem):
  sid = lax.axis_index("s")
  wid = sid * NC + lax.axis_index("c")
  acc = acc_sh

  pltpu.sync_copy(w_hbm.at[wid], widx)
  wbase = wid * ROWS_PER_W
  pieces = ((table_hbm.at[:, pl.ds(0, 128)], embw0_hbm, wbuf0),
            (table_hbm.at[:, pl.ds(128, 128)], embw1_hbm, wbuf1),
            (ttail_hbm, embw2_hbm, wbuf2))
  for k in range(NWCHUNK):
    rows = pl.ds(wbase + k * WCHUNK, WCHUNK)
    for src, out_hbm, buf in pieces:
      pltpu.async_copy(src.at[widx.at[k]], buf, sem).wait()
      pltpu.sync_copy(buf, out_hbm.at[rows])

  zseg = jnp.zeros((16,), _f32)

  def _zero_row(i, _):
    for q in range(DCP // 16):
      zbuf[i, pl.ds(q * 16, 16)] = zseg
    return 0

  lax.fori_loop(0, ZROWS, _zero_row, 0)

  cid = lax.axis_index("c")
  myrows = pl.ds(sid * ZROWS, ZROWS)

  def _row(j, _):
    b = cid * (B // NC) + j
    pltpu.sync_copy(c_hbm.at[b, sid], cbuf)
    pltpu.sync_copy(p_hbm.at[b, sid], pbuf)

    for q in range(CCHUNK // 16):
      v = cbuf[pl.ds(q * 16, 16)]
      v = jnp.where(v == 32, 0, v)
      v = jnp.minimum(v, 256)
      cbuf[pl.ds(q * 16, 16)] = v

    pltpu.sync_copy(zbuf, acc.at[myrows])
    plsc.subcore_barrier()
    pltpu.async_copy(ctab_hbm.at[cbuf], gbuf, sem).wait()
    pltpu.sync_copy(gbuf, acc.at[pbuf], add=True)
    plsc.subcore_barrier()
    pltpu.sync_copy(acc.at[myrows], acc_hbm.at[b, myrows])
    plsc.subcore_barrier()
    return 0

  lax.fori_loop(0, B // NC, _row, 0)


_sc_call = functools.partial(
    pl.kernel,
    out_type=(
        jax.ShapeDtypeStruct((B * LW, 128), _f32),
        jax.ShapeDtypeStruct((B * LW, 128), _f32),
        jax.ShapeDtypeStruct((B * LW, 128), _f32),
        jax.ShapeDtypeStruct((B, LW, DCP), _f32),
    ),
    mesh=plsc.VectorSubcoreMesh(core_axis_name="c", subcore_axis_name="s"),
    scratch_types=[
        pltpu.VMEM((NWCHUNK, WCHUNK), _i32),
        pltpu.VMEM((WCHUNK, 128), _f32),
        pltpu.VMEM((WCHUNK, 128), _f32),
        pltpu.VMEM((WCHUNK, 128), _f32),
        pltpu.VMEM((CCHUNK,), _i32),
        pltpu.VMEM((CCHUNK,), _i32),
        pltpu.VMEM((CCHUNK, DCP), _f32),
        pltpu.VMEM((ZROWS, DCP), _f32),
        pltpu.VMEM_SHARED((LW, DCP), _f32),
        pltpu.SemaphoreType.DMA,
    ],
)(_sc_body)


def _tc_body(e0_ref, e1_ref, e2_ref, acc_ref, wp0_ref, wp1_ref, wp2_ref,
             wpc_ref,
             wg0_ref, bg0_ref, wt0_ref, bt0_ref,
             wg1_ref, bg1_ref, wt1_ref, bt1_ref, out_ref):
  a = acc_ref[...]
  cnt = a[:, DC:DC + 1]
  scale = 1.0 / (0.001 + jnp.sqrt(cnt))
  ec = a[:, :DC] * scale
  x = jnp.dot(e0_ref[...], wp0_ref[...], preferred_element_type=_f32)
  x = x + jnp.dot(e1_ref[...], wp1_ref[...], preferred_element_type=_f32)
  x = x + jnp.dot(e2_ref[...], wp2_ref[...], preferred_element_type=_f32)
  x = x + jnp.dot(ec, wpc_ref[...], preferred_element_type=_f32)
  for wg, bg, wt, bt in ((wg0_ref, bg0_ref, wt0_ref, bt0_ref),
                         (wg1_ref, bg1_ref, wt1_ref, bt1_ref)):
    g = jax.nn.sigmoid(jnp.dot(x, wg[...], preferred_element_type=_f32)
                       + bg[...])
    t = jax.nn.relu(jnp.dot(x, wt[...], preferred_element_type=_f32)
                    + bt[...])
    x = g * t + (1.0 - g) * x
  out_ref[...] = x


ROWB = 512
_N_BLK = B * LW // ROWB

_w_spec = pl.BlockSpec((H, H), lambda i: (0, 0))
_b_spec = pl.BlockSpec((1, H), lambda i: (0, 0))

_tc_call = pl.pallas_call(
    _tc_body,
    grid=(_N_BLK,),
    in_specs=[
        pl.BlockSpec((ROWB, 128), lambda i: (i, 0)),
        pl.BlockSpec((ROWB, 128), lambda i: (i, 0)),
        pl.BlockSpec((ROWB, 128), lambda i: (i, 0)),
        pl.BlockSpec((ROWB, DCP), lambda i: (i, 0)),
        pl.BlockSpec((128, H), lambda i: (0, 0)),
        pl.BlockSpec((128, H), lambda i: (0, 0)),
        pl.BlockSpec((128, H), lambda i: (0, 0)),
        pl.BlockSpec((DC, H), lambda i: (0, 0)),
        _w_spec, _b_spec, _w_spec, _b_spec,
        _w_spec, _b_spec, _w_spec, _b_spec,
    ],
    out_specs=pl.BlockSpec((ROWB, H), lambda i: (i, 0)),
    out_shape=jax.ShapeDtypeStruct((B * LW, H), _f32),
)


def kernel(w, c, p, word_vectors, char_table, W_proj,
           Wt0, bt0, Wg0, bg0, Wt1, bt1, Wg1, bg1):
  w3 = w.astype(_i32).reshape(NW, NWCHUNK, WCHUNK)
  c3 = c.astype(_i32).reshape(B, NCCHUNK, CCHUNK)
  p3 = p.astype(_i32).reshape(B, NCCHUNK, CCHUNK)
  ctab = jnp.concatenate(
      [char_table.astype(_f32),
       jnp.ones((257, 1), _f32),
       jnp.zeros((257, DCP - DC - 1), _f32)], axis=1)

  wv = word_vectors.astype(_f32)
  ttail = jnp.pad(wv[:, 256:DW], ((0, 0), (0, 128 - (DW - 256))))
  e0, e1, e2, acc = _sc_call(w3, c3, p3, wv, ttail, ctab)

  x = _tc_call(
      e0, e1, e2, acc.reshape(B * LW, DCP),
      W_proj[0:128], W_proj[128:256],
      jnp.pad(W_proj[256:DW], ((0, 128 - (DW - 256)), (0, 0))), W_proj[DW:],
      Wg0, bg0.reshape(1, H), Wt0, bt0.reshape(1, H),
      Wg1, bg1.reshape(1, H), Wt1, bt1.reshape(1, H))
  return x.reshape(B, LW, H)
